# baseline (speedup 1.0000x reference)
import functools

import jax
import jax.numpy as jnp
from jax import lax
from jax.experimental import pallas as pl
from jax.experimental.pallas import tpu as pltpu
from jax.experimental.pallas import tpu_sc as plsc

NUM_FIELDS = 26
VOCAB = 100000
EMB = 64
BATCH = 4096

NC, NS = 2, 16
NW = NC * NS
CHUNK = BATCH // NW
NF_PAD = 32


def _gather_body(tab_hbm, idx_hbm, out_hbm, idx_v, rows_v, gsem):
    wid = lax.axis_index("s") * NC + lax.axis_index("c")
    pltpu.sync_copy(idx_hbm.at[wid], idx_v)

    for j in range(NUM_FIELDS):
        pltpu.async_copy(tab_hbm.at[j].at[idx_v.at[j]], rows_v, gsem).wait()
        pltpu.sync_copy(rows_v, out_hbm.at[pl.ds(wid * CHUNK, CHUNK), j])


@jax.jit
def _gather(tables, idx3d):
    mesh = plsc.VectorSubcoreMesh(core_axis_name="c", subcore_axis_name="s")
    f = pl.kernel(
        _gather_body,
        out_type=jax.ShapeDtypeStruct((BATCH, NUM_FIELDS, EMB), jnp.float32),
        mesh=mesh,
        scratch_types=[
            pltpu.VMEM((NF_PAD, CHUNK), jnp.int32),
            pltpu.VMEM((CHUNK, EMB), jnp.float32),
            pltpu.SemaphoreType.DMA,
        ],
        compiler_params=pltpu.CompilerParams(use_tc_tiling_on_sc=False),
    )
    return f(tables, idx3d)


def kernel(x, tables):
    idx3d = x.reshape(NW, CHUNK, NUM_FIELDS).transpose(0, 2, 1)
    idx3d = jnp.pad(idx3d, ((0, 0), (0, NF_PAD - NUM_FIELDS), (0, 0)))
    out = _gather(tables, idx3d)
    return out.reshape(BATCH, NUM_FIELDS * EMB)
